# baseline (speedup 1.0000x reference)
import functools

import jax
import jax.numpy as jnp
from jax import lax
from jax.experimental import pallas as pl
from jax.experimental.pallas import tpu as pltpu
from jax.experimental.pallas import tpu_sc as plsc

B = 16384
F = 64
H = 100
NC = 2
NS = 16
NW = NC * NS
ROWS_W = B // NW
CHUNK = 128
NCH = ROWS_W // CHUNK


def _sc_gather_one(tab, bias1, idx):
  mesh = plsc.VectorSubcoreMesh(core_axis_name="c", subcore_axis_name="s")

  @functools.partial(
      pl.kernel,
      mesh=mesh,
      out_type=[
          jax.ShapeDtypeStruct((B, F), jnp.float32),
          jax.ShapeDtypeStruct((B,), jnp.float32),
      ],
      scratch_types=[
          pltpu.VMEM((NCH, CHUNK), jnp.int32),
          pltpu.VMEM((CHUNK, F), jnp.float32),
          pltpu.VMEM((CHUNK, F), jnp.float32),
          pltpu.VMEM((ROWS_W,), jnp.float32),
          pltpu.SemaphoreType.DMA,
          pltpu.SemaphoreType.DMA,
          pltpu.SemaphoreType.DMA,
      ],
      compiler_params=pltpu.CompilerParams(use_tc_tiling_on_sc=False),
  )
  def k(tab_h, bias_h, idx_h, rows_out, bias_out,
        idx_v, buf_a, buf_b, bias_v, sem_a, sem_b, sem_c):
    wid = lax.axis_index("s") * NC + lax.axis_index("c")
    base = wid * ROWS_W
    pltpu.sync_copy(idx_h.at[wid], idx_v)
    bias_copies = []
    for j in range(NCH):
      r = pl.ds(j * CHUNK, CHUNK)
      bias_copies.append(
          pltpu.async_copy(bias_h.at[idx_v.at[j]], bias_v.at[r], sem_c))
    bufs = [(buf_a, sem_a), (buf_b, sem_b)]

    def issue(j):
      buf, sem = bufs[j % 2]
      return pltpu.async_copy(tab_h.at[idx_v.at[j]], buf, sem)

    cur = issue(0)
    for j in range(NCH):
      nxt = issue(j + 1) if j + 1 < NCH else None
      cur.wait()
      buf, _ = bufs[j % 2]
      pltpu.sync_copy(buf, rows_out.at[pl.ds(base + j * CHUNK, CHUNK)])
      cur = nxt
    for c in bias_copies:
      c.wait()
    pltpu.sync_copy(bias_v, bias_out.at[pl.ds(base, ROWS_W)])

  return k(tab, bias1, idx)


BLK = 2048


def _tc_body(u_r, it_r, ub_r, ib_r,
             w1u_r, w1ub_r, w1i_r, w1ib_r, b1_r,
             w2_r, b2_r, w3_r, b3_r, w4_r, b4_r, sd_r, out_r):
  u = u_r[...]
  it = it_r[...]
  ub = ub_r[...]
  ib = ib_r[...]
  sd = jnp.sum(u * it, axis=1, keepdims=True) + ub + ib
  h = jnp.dot(u, w1u_r[...], preferred_element_type=jnp.float32)
  h = h + jnp.dot(it, w1i_r[...], preferred_element_type=jnp.float32)
  h = h + ub * w1ub_r[...] + ib * w1ib_r[...] + b1_r[...]
  h = jnp.maximum(h, 0.0)
  h = jnp.dot(h, w2_r[...], preferred_element_type=jnp.float32) + b2_r[...]
  h = jnp.maximum(h, 0.0)
  h = jnp.dot(h, w3_r[...], preferred_element_type=jnp.float32) + b3_r[...] + sd
  out = jnp.dot(h, w4_r[...], preferred_element_type=jnp.float32) + b4_r[...]
  sd_r[...] = sd
  out_r[...] = out


def _tc_mlp(u, it, ub, ib, w1u, w1ub, w1i, w1ib, b1, w2, b2, w3, b3, w4, b4):
  full = lambda shape: pl.BlockSpec(shape, lambda i: (0, 0))
  rows = lambda shape: pl.BlockSpec(shape, lambda i: (i, 0))
  return pl.pallas_call(
      _tc_body,
      grid=(B // BLK,),
      in_specs=[
          rows((BLK, F)), rows((BLK, F)), rows((BLK, 1)), rows((BLK, 1)),
          full((F, H)), full((1, H)), full((F, H)), full((1, H)), full((1, H)),
          full((H, H)), full((1, H)), full((H, H)), full((1, H)),
          full((H, 1)), full((1, 1)),
      ],
      out_specs=[rows((BLK, 1)), rows((BLK, 1))],
      out_shape=[
          jax.ShapeDtypeStruct((B, 1), jnp.float32),
          jax.ShapeDtypeStruct((B, 1), jnp.float32),
      ],
  )(u, it, ub, ib, w1u, w1ub, w1i, w1ib, b1, w2, b2, w3, b3, w4, b4)


def kernel(item_in, user_in, user_factors, user_bias, item_factors, item_bias,
           W1, b1, W2, b2, W3, b3, W4, b4):
  uidx = user_in.reshape(NW, NCH, CHUNK)
  iidx = item_in.reshape(NW, NCH, CHUNK)
  ub1 = user_bias.reshape(-1)
  ib1 = item_bias.reshape(-1)
  ug, ubg = _sc_gather_one(user_factors, ub1, uidx)
  itg, ibg = _sc_gather_one(item_factors, ib1, iidx)
  w1u = W1[0:F]
  w1ub = W1[F:F + 1]
  w1i = W1[F + 1:2 * F + 1]
  w1ib = W1[2 * F + 1:2 * F + 2]
  sd, out = _tc_mlp(ug, itg, ubg.reshape(B, 1), ibg.reshape(B, 1),
                    w1u, w1ub, w1i, w1ib, b1.reshape(1, H),
                    W2, b2.reshape(1, H), W3, b3.reshape(1, H), W4,
                    b4.reshape(1, 1))
  return (sd, out)
